# baseline (device time: 74665 ns/iter reference)
import jax
import jax.numpy as jnp
from jax import lax
from jax.experimental import pallas as pl
from jax.experimental.pallas import tpu as pltpu

N_DEV = 4
B_PER = 2
SQ = 128
D = 512
H_PER = 8
DH = 64
ROWS = B_PER * SQ


def kernel(x, Wq, Wo, Wk, Wv):
    def body(x_ref, wq_ref, wo_ref, wk_ref, wv_ref, out_ref,
             xall, contribs, recvbuf, attn_ref,
             ag_send, ag_recv, rs_send, rs_recv):
        my = lax.axis_index("i")
        left = lax.rem(my + N_DEV - 1, N_DEV)
        right = lax.rem(my + 1, N_DEV)

        barrier = pltpu.get_barrier_semaphore()
        for nbr in (left, right):
            pl.semaphore_signal(
                barrier, inc=1, device_id=(nbr,),
                device_id_type=pl.DeviceIdType.MESH,
            )
        pl.semaphore_wait(barrier, 2)

        xall[0] = x_ref[...]

        for h in range(N_DEV - 1):
            rdma = pltpu.make_async_remote_copy(
                src_ref=xall.at[h],
                dst_ref=xall.at[h + 1],
                send_sem=ag_send.at[h],
                recv_sem=ag_recv.at[h],
                device_id=(right,),
                device_id_type=pl.DeviceIdType.MESH,
            )
            rdma.start()
            rdma.wait()

        def contribution(r):
            xc = xall[r]
            q = jnp.dot(xc, wq_ref[...], preferred_element_type=jnp.float32)
            k = jnp.dot(xc, wk_ref[...], preferred_element_type=jnp.float32)
            v = jnp.dot(xc, wv_ref[...], preferred_element_type=jnp.float32)
            for b in range(B_PER):
                rsl = slice(b * SQ, (b + 1) * SQ)
                for hh in range(H_PER):
                    csl = slice(hh * DH, (hh + 1) * DH)
                    qh = q[rsl, csl]
                    kh = k[rsl, csl]
                    vh = v[rsl, csl]
                    s = lax.dot_general(
                        qh, kh, (((1,), (1,)), ((), ())),
                        preferred_element_type=jnp.float32,
                    ) * 0.125
                    m = jnp.max(s, axis=-1, keepdims=True)
                    p = jnp.exp(s - m)
                    lsum = jnp.sum(p, axis=-1, keepdims=True)
                    o = jnp.dot(p, vh, preferred_element_type=jnp.float32) / lsum
                    attn_ref[rsl, csl] = o
            return jnp.dot(
                attn_ref[...], wo_ref[...], preferred_element_type=jnp.float32
            )

        for r in range(N_DEV):
            contribs[r] = contribution(r)

        for s in range(N_DEV - 1):
            rdma = pltpu.make_async_remote_copy(
                src_ref=contribs.at[s + 1],
                dst_ref=recvbuf.at[s],
                send_sem=rs_send.at[s],
                recv_sem=rs_recv.at[s],
                device_id=(right,),
                device_id_type=pl.DeviceIdType.MESH,
            )
            rdma.start()
            rdma.wait()
            if s < N_DEV - 2:
                contribs[s + 2] = contribs[s + 2] + recvbuf[s]

        out_ref[...] = contribs[0] + recvbuf[N_DEV - 2]

    x2 = x.reshape(ROWS, D)
    out = pl.pallas_call(
        body,
        out_shape=jax.ShapeDtypeStruct((ROWS, D), jnp.float32),
        in_specs=[pl.BlockSpec(memory_space=pltpu.VMEM)] * 5,
        out_specs=pl.BlockSpec(memory_space=pltpu.VMEM),
        scratch_shapes=[
            pltpu.VMEM((N_DEV, ROWS, D), jnp.float32),
            pltpu.VMEM((N_DEV, ROWS, D), jnp.float32),
            pltpu.VMEM((N_DEV - 1, ROWS, D), jnp.float32),
            pltpu.VMEM((ROWS, D), jnp.float32),
            pltpu.SemaphoreType.DMA((N_DEV - 1,)),
            pltpu.SemaphoreType.DMA((N_DEV - 1,)),
            pltpu.SemaphoreType.DMA((N_DEV - 1,)),
            pltpu.SemaphoreType.DMA((N_DEV - 1,)),
        ],
        compiler_params=pltpu.CompilerParams(collective_id=0),
    )(x2, Wq, Wo, Wk, Wv)
    return out.reshape(B_PER, SQ, D)


# device time: 48980 ns/iter; 1.5244x vs baseline; 1.5244x over previous
import jax
import jax.numpy as jnp
from jax import lax
from jax.experimental import pallas as pl
from jax.experimental.pallas import tpu as pltpu

N_DEV = 4
B_PER = 2
SQ = 128
D = 512
H_PER = 8
DH = 64
ROWS = B_PER * SQ


def kernel(x, Wq, Wo, Wk, Wv):
    def body(x_ref, wq_ref, wo_ref, wk_ref, wv_ref, out_ref,
             xall, contribs, recvbuf, attn_ref,
             ag_send, ag_recv, rs_send, rs_recv):
        my = lax.axis_index("i")
        left = lax.rem(my + N_DEV - 1, N_DEV)
        right = lax.rem(my + 1, N_DEV)

        barrier = pltpu.get_barrier_semaphore()
        for nbr in (left, right):
            pl.semaphore_signal(
                barrier, inc=1, device_id=(nbr,),
                device_id_type=pl.DeviceIdType.MESH,
            )
        pl.semaphore_wait(barrier, 2)

        xall[0] = x_ref[...]

        def ag_hop(h):
            return pltpu.make_async_remote_copy(
                src_ref=xall.at[h],
                dst_ref=xall.at[h + 1],
                send_sem=ag_send.at[h],
                recv_sem=ag_recv.at[h],
                device_id=(right,),
                device_id_type=pl.DeviceIdType.MESH,
            )

        def rs_step(s):
            return pltpu.make_async_remote_copy(
                src_ref=contribs.at[s + 1],
                dst_ref=recvbuf.at[s],
                send_sem=rs_send.at[s],
                recv_sem=rs_recv.at[s],
                device_id=(right,),
                device_id_type=pl.DeviceIdType.MESH,
            )

        def contribution(r):
            xc = xall[r]
            q = jnp.dot(xc, wq_ref[...], preferred_element_type=jnp.float32)
            k = jnp.dot(xc, wk_ref[...], preferred_element_type=jnp.float32)
            v = jnp.dot(xc, wv_ref[...], preferred_element_type=jnp.float32)
            for b in range(B_PER):
                rsl = slice(b * SQ, (b + 1) * SQ)
                for hh in range(H_PER):
                    csl = slice(hh * DH, (hh + 1) * DH)
                    qh = q[rsl, csl]
                    kh = k[rsl, csl]
                    vh = v[rsl, csl]
                    s = lax.dot_general(
                        qh, kh, (((1,), (1,)), ((), ())),
                        preferred_element_type=jnp.float32,
                    ) * 0.125
                    m = jnp.max(s, axis=-1, keepdims=True)
                    p = jnp.exp(s - m)
                    lsum = jnp.sum(p, axis=-1, keepdims=True)
                    o = jnp.dot(p, vh, preferred_element_type=jnp.float32) / lsum
                    attn_ref[rsl, csl] = o
            return jnp.dot(
                attn_ref[...], wo_ref[...], preferred_element_type=jnp.float32
            )

        ag0 = ag_hop(0)
        ag0.start()
        contribs[0] = contribution(0)

        ag0.wait_recv()
        ag1 = ag_hop(1)
        ag1.start()
        contribs[1] = contribution(1)

        rs0 = rs_step(0)
        rs0.start()
        ag1.wait_recv()
        ag2 = ag_hop(2)
        ag2.start()
        contribs[2] = contribution(2)

        rs0.wait_recv()
        contribs[2] = contribs[2] + recvbuf[0]
        rs1 = rs_step(1)
        rs1.start()
        ag2.wait_recv()
        contribs[3] = contribution(3)

        rs1.wait_recv()
        contribs[3] = contribs[3] + recvbuf[1]
        rs2 = rs_step(2)
        rs2.start()
        rs2.wait_recv()
        out_ref[...] = contribs[0] + recvbuf[N_DEV - 2]

        for d in (ag0, ag1, ag2, rs0, rs1, rs2):
            d.wait_send()

    x2 = x.reshape(ROWS, D)
    out = pl.pallas_call(
        body,
        out_shape=jax.ShapeDtypeStruct((ROWS, D), jnp.float32),
        in_specs=[pl.BlockSpec(memory_space=pltpu.VMEM)] * 5,
        out_specs=pl.BlockSpec(memory_space=pltpu.VMEM),
        scratch_shapes=[
            pltpu.VMEM((N_DEV, ROWS, D), jnp.float32),
            pltpu.VMEM((N_DEV, ROWS, D), jnp.float32),
            pltpu.VMEM((N_DEV - 1, ROWS, D), jnp.float32),
            pltpu.VMEM((ROWS, D), jnp.float32),
            pltpu.SemaphoreType.DMA((N_DEV - 1,)),
            pltpu.SemaphoreType.DMA((N_DEV - 1,)),
            pltpu.SemaphoreType.DMA((N_DEV - 1,)),
            pltpu.SemaphoreType.DMA((N_DEV - 1,)),
        ],
        compiler_params=pltpu.CompilerParams(collective_id=0),
    )(x2, Wq, Wo, Wk, Wv)
    return out.reshape(B_PER, SQ, D)
